# baseline (speedup 1.0000x reference)
import functools

import jax
import jax.numpy as jnp
from jax import lax
from jax.experimental import pallas as pl
from jax.experimental.pallas import tpu as pltpu
from jax.experimental.pallas import tpu_sc as plsc

N = 10000
E = 320000
D = 128
NCLS = 40

NC = 2
NS = 16
K = 128
CHT = 160
E_PAD = NS * CHT * K
R_PAD = 10240
HALF = R_PAD // NC
ACC_R = HALF + K
ZR = 64
RPT = HALF // NS
NBUF = 4
BINCH = 96
BCAP = BINCH * K
BCH = 48
SCANB = 40


def _bin_body(rows_hbm, cols_hbm, brow_hbm, bcol_hbm,
              row_v, col_v, brow_b, bcol_b):
    c = lax.axis_index("c")
    s = lax.axis_index("s")
    lanes = lax.iota(jnp.int32, 16)

    def _pf(i, _):
        bcol_b[pl.ds(i * 16, 16)] = HALF + lax.rem(i, 8) * 16 + lanes
        brow_b[pl.ds(i * 16, 16)] = jnp.full((16,), N, jnp.int32)
        return 0
    lax.fori_loop(0, BCAP // 16, _pf, 0)

    def _batch(t, off):
        pltpu.sync_copy(rows_hbm.at[s, pl.ds(t * SCANB * K, SCANB * K)],
                        row_v)
        pltpu.sync_copy(cols_hbm.at[s, pl.ds(t * SCANB * K, SCANB * K)],
                        col_v)

        def _vec(i, off):
            cv = col_v[pl.ds(i * 16, 16)]
            rv = row_v[pl.ds(i * 16, 16)]
            loc = cv - c * HALF
            msk = (loc >= 0) & (loc < HALF)
            key = jnp.where(msk, 0, 16) + lanes
            colsafe = jnp.where(msk, loc, HALF + lanes)
            rowsafe = jnp.where(msk, rv, N)
            bcol_b[pl.ds(off, 16)] = plsc.sort_key_val(key, colsafe)[1]
            brow_b[pl.ds(off, 16)] = plsc.sort_key_val(key, rowsafe)[1]
            n = plsc.all_reduce_population_count(msk)
            return lax.min(off + jnp.max(n), BCAP)
        return lax.fori_loop(0, SCANB * K // 16, _vec, off)
    lax.fori_loop(0, CHT // SCANB, _batch, 0)

    pltpu.sync_copy(brow_b.at[pl.ds(0, BCAP)], brow_hbm.at[c, s])
    pltpu.sync_copy(bcol_b.at[pl.ds(0, BCAP)], bcol_hbm.at[c, s])


def _deg_body(colsr_hbm, out_hbm, col_v, ones_v, zbuf, acc):
    c = lax.axis_index("c")
    s = lax.axis_index("s")
    wid = c * NS + s

    def _fill(r, _):
        def _fj(j, _):
            ones_v[r, pl.ds(j * 16, 16)] = jnp.ones((16,), jnp.float32)
            zbuf[lax.rem(r, ZR), pl.ds(j * 16, 16)] = jnp.zeros(
                (16,), jnp.float32)
            return 0
        return lax.fori_loop(0, D // 16, _fj, 0)
    lax.fori_loop(0, K, _fill, 0)

    def _zc(k, _):
        pltpu.sync_copy(zbuf, acc.at[pl.ds(s * RPT + k * ZR, ZR)])
        return 0
    lax.fori_loop(0, RPT // ZR, _zc, 0)

    pltpu.sync_copy(colsr_hbm.at[wid], col_v)
    plsc.subcore_barrier()

    def _step(i, _):
        pltpu.sync_copy(ones_v, acc.at[col_v.at[i]], add=True)
        return 0
    lax.fori_loop(0, BINCH, _step, 0)

    plsc.subcore_barrier()
    pltpu.sync_copy(acc.at[pl.ds(s * RPT, RPT)],
                    out_hbm.at[c, pl.ds(s * RPT, RPT)])


def _scatter_body(y_hbm, rows_hbm, colsr_hbm, out_hbm,
                  row_v, col_v, gbuf, zbuf, acc, sem):
    c = lax.axis_index("c")
    s = lax.axis_index("s")
    wid = c * NS + s

    def _zr(r, _):
        def _zj(j, _):
            zbuf[r, pl.ds(j * 16, 16)] = jnp.zeros((16,), jnp.float32)
            return 0
        return lax.fori_loop(0, D // 16, _zj, 0)
    lax.fori_loop(0, ZR, _zr, 0)

    def _zc(k, _):
        pltpu.sync_copy(zbuf, acc.at[pl.ds(s * RPT + k * ZR, ZR)])
        return 0
    lax.fori_loop(0, RPT // ZR, _zc, 0)

    plsc.subcore_barrier()

    def _batch(t, _):
        pltpu.sync_copy(rows_hbm.at[wid, pl.ds(t * BCH, BCH)], row_v)
        pltpu.sync_copy(colsr_hbm.at[wid, pl.ds(t * BCH, BCH)], col_v)

        def _grp(g, _):
            def _fire(b, _):
                pltpu.async_copy(
                    y_hbm.at[row_v.at[g * NBUF + b]], gbuf.at[b], sem)
                return 0
            lax.fori_loop(0, NBUF, _fire, 0)

            def _drain(b, _):
                pltpu.make_async_copy(
                    y_hbm.at[row_v.at[g * NBUF + b]], gbuf.at[b], sem).wait()
                return 0
            lax.fori_loop(0, NBUF, _drain, 0)

            def _scat(b, _):
                pltpu.sync_copy(gbuf.at[b], acc.at[col_v.at[g * NBUF + b]],
                                add=True)
                return 0
            lax.fori_loop(0, NBUF, _scat, 0)
            return 0
        lax.fori_loop(0, BCH // NBUF, _grp, 0)
        return 0
    lax.fori_loop(0, BINCH // BCH, _batch, 0)

    plsc.subcore_barrier()
    pltpu.sync_copy(acc.at[pl.ds(s * RPT, RPT)],
                    out_hbm.at[c, pl.ds(s * RPT, RPT)])


@functools.lru_cache(maxsize=1)
def _sc_kernels():
    mesh = plsc.VectorSubcoreMesh(
        core_axis_name="c", subcore_axis_name="s",
        num_cores=NC, num_subcores=NS)
    bin_k = pl.kernel(
        _bin_body,
        out_type=[jax.ShapeDtypeStruct((NC, NS, BCAP), jnp.int32),
                  jax.ShapeDtypeStruct((NC, NS, BCAP), jnp.int32)],
        mesh=mesh,
        compiler_params=pltpu.CompilerParams(needs_layout_passes=False),
        scratch_types=[
            pltpu.VMEM((SCANB * K,), jnp.int32),
            pltpu.VMEM((SCANB * K,), jnp.int32),
            pltpu.VMEM((BCAP + 16,), jnp.int32),
            pltpu.VMEM((BCAP + 16,), jnp.int32),
        ],
    )
    deg_k = pl.kernel(
        _deg_body,
        out_type=jax.ShapeDtypeStruct((NC, HALF, D), jnp.float32),
        mesh=mesh,
        scratch_types=[
            pltpu.VMEM((BINCH, K), jnp.int32),
            pltpu.VMEM((K, D), jnp.float32),
            pltpu.VMEM((ZR, D), jnp.float32),
            pltpu.VMEM_SHARED((ACC_R, D), jnp.float32),
        ],
    )
    scat_k = pl.kernel(
        _scatter_body,
        out_type=jax.ShapeDtypeStruct((NC, HALF, D), jnp.float32),
        mesh=mesh,
        scratch_types=[
            pltpu.VMEM((BCH, K), jnp.int32),
            pltpu.VMEM((BCH, K), jnp.int32),
            pltpu.VMEM((NBUF, K, D), jnp.float32),
            pltpu.VMEM((ZR, D), jnp.float32),
            pltpu.VMEM_SHARED((ACC_R, D), jnp.float32),
            pltpu.SemaphoreType.DMA,
        ],
    )
    return bin_k, deg_k, scat_k


def _k1_body(deg_ref, x_ref, w1_ref, y1_ref, dinvb_ref):
    cnt = jnp.max(deg_ref[...], axis=1, keepdims=True)
    dinvb = jnp.broadcast_to(lax.rsqrt(cnt + 1.0), (D, D))
    xw = jnp.dot(x_ref[...], w1_ref[...], preferred_element_type=jnp.float32)
    y1_ref[...] = dinvb * xw
    dinvb_ref[...] = dinvb


def _k2_body(acc_ref, y1_ref, dinvb_ref, w2_ref, b1_ref, y2_ref):
    agg = acc_ref[...] + y1_ref[...]
    h1 = jnp.maximum(dinvb_ref[...] * agg + b1_ref[...], 0.0)
    y2_ref[...] = dinvb_ref[...] * jnp.dot(
        h1, w2_ref[...], preferred_element_type=jnp.float32)


def _k3_body(acc_ref, y2_ref, dinvb_ref, b2_ref, w3_ref, b3_ref, out_ref):
    agg = acc_ref[...] + y2_ref[...]
    h2 = jnp.maximum(dinvb_ref[...] * agg + b2_ref[...], 0.0)
    logits = jnp.dot(h2, w3_ref[...],
                     preferred_element_type=jnp.float32) + b3_ref[...]
    m = jnp.max(logits, axis=1, keepdims=True)
    e = jnp.exp(logits - m)
    out_ref[...] = e / jnp.sum(e, axis=1, keepdims=True)


_G = R_PAD // D

_blk_rows = pl.BlockSpec((D, D), lambda i: (i, 0))
_blk_full = pl.BlockSpec((D, D), lambda i: (0, 0))
_blk_bias = pl.BlockSpec((1, D), lambda i: (0, 0))

_k1 = pl.pallas_call(
    _k1_body,
    grid=(_G,),
    in_specs=[_blk_rows, _blk_rows, _blk_full],
    out_specs=[_blk_rows, _blk_rows],
    out_shape=[jax.ShapeDtypeStruct((R_PAD, D), jnp.float32),
               jax.ShapeDtypeStruct((R_PAD, D), jnp.float32)],
)

_k2 = pl.pallas_call(
    _k2_body,
    grid=(_G,),
    in_specs=[_blk_rows, _blk_rows, _blk_rows, _blk_full, _blk_bias],
    out_specs=_blk_rows,
    out_shape=jax.ShapeDtypeStruct((R_PAD, D), jnp.float32),
)

_k3 = pl.pallas_call(
    _k3_body,
    grid=(_G,),
    in_specs=[_blk_rows, _blk_rows, _blk_rows, _blk_bias, _blk_full,
              _blk_bias],
    out_specs=_blk_rows,
    out_shape=jax.ShapeDtypeStruct((R_PAD, D), jnp.float32),
)


def kernel(x, edge_index, W1, b1, W2, b2, W3, b3):
    row = edge_index[0].astype(jnp.int32)
    col = edge_index[1].astype(jnp.int32)
    pad_r = jnp.full((E_PAD - E,), N, jnp.int32)
    pad_c = jnp.full((E_PAD - E,), -1, jnp.int32)
    rows3 = jnp.concatenate([row, pad_r]).reshape(NS, CHT * K)
    cols3 = jnp.concatenate([col, pad_c]).reshape(NS, CHT * K)

    x_pad = jnp.concatenate(
        [x, jnp.zeros((R_PAD - N, D), jnp.float32)], axis=0)
    b1r = b1.reshape(1, D)
    b2r = b2.reshape(1, D)
    W3p = jnp.concatenate(
        [W3, jnp.zeros((D, D - NCLS), jnp.float32)], axis=1)
    b3p = jnp.concatenate(
        [b3, jnp.full((D - NCLS,), -1e30, jnp.float32)]).reshape(1, D)

    bin_kernel, deg_kernel, scatter_kernel = _sc_kernels()
    brows, bcols = bin_kernel(rows3, cols3)
    brows = brows.reshape(NC * NS, BINCH, K)
    bcols = bcols.reshape(NC * NS, BINCH, K)
    deg = deg_kernel(bcols).reshape(R_PAD, D)
    y1, dinvb = _k1(deg, x_pad, W1)
    acc1 = scatter_kernel(y1, brows, bcols).reshape(R_PAD, D)
    y2 = _k2(acc1, y1, dinvb, W2, b1r)
    acc2 = scatter_kernel(y2, brows, bcols).reshape(R_PAD, D)
    probs = _k3(acc2, y2, dinvb, b2r, W3p, b3p)
    return probs[:N, :NCLS]


# speedup vs baseline: 9.3976x; 9.3976x over previous
import functools

import jax
import jax.numpy as jnp
from jax import lax
from jax.experimental import pallas as pl
from jax.experimental.pallas import tpu as pltpu
from jax.experimental.pallas import tpu_sc as plsc

N = 10000
E = 320000
D = 128
NCLS = 40

NC = 2
NS = 16
K = 128
CHT = 160
E_PAD = NS * CHT * K
R_PAD = 10240
HALF = R_PAD // NC
ACC_R = HALF + K
ZR = 64
RPT = HALF // NS
NBUF = 4
BINCH = 96
BCAP = BINCH * K
BCH = 48
SCANB = 40


def _bin_body(rows_hbm, cols_hbm, brow_hbm, bcol_hbm,
              row_v, col_v, brow_b, bcol_b):
    c = lax.axis_index("c")
    s = lax.axis_index("s")
    lanes = lax.iota(jnp.int32, 16)

    def _pf(i, _):
        bcol_b[pl.ds(i * 16, 16)] = HALF + lax.rem(i, 8) * 16 + lanes
        brow_b[pl.ds(i * 16, 16)] = N + 16 + lax.rem(i, 14) * 16 + lanes
        return 0
    lax.fori_loop(0, BCAP // 16, _pf, 0)

    def _batch(t, off):
        pltpu.sync_copy(rows_hbm.at[s, pl.ds(t * SCANB * K, SCANB * K)],
                        row_v)
        pltpu.sync_copy(cols_hbm.at[s, pl.ds(t * SCANB * K, SCANB * K)],
                        col_v)

        def _vec(i, off):
            cv = col_v[pl.ds(i * 16, 16)]
            rv = row_v[pl.ds(i * 16, 16)]
            loc = cv - c * HALF
            msk = (loc >= 0) & (loc < HALF)
            key = jnp.where(msk, 0, 16) + lanes
            colsafe = jnp.where(msk, loc, HALF + lanes)
            rowsafe = jnp.where(msk, rv, N)
            bcol_b[pl.ds(off, 16)] = plsc.sort_key_val(key, colsafe)[1]
            brow_b[pl.ds(off, 16)] = plsc.sort_key_val(key, rowsafe)[1]
            n = plsc.all_reduce_population_count(msk)
            return lax.min(off + jnp.max(n), BCAP)
        return lax.fori_loop(0, SCANB * K // 16, _vec, off)
    lax.fori_loop(0, CHT // SCANB, _batch, 0)

    pltpu.sync_copy(brow_b.at[pl.ds(0, BCAP)], brow_hbm.at[c, s])
    pltpu.sync_copy(bcol_b.at[pl.ds(0, BCAP)], bcol_hbm.at[c, s])


def _deg_body(colsr_hbm, out_hbm, col_v, ones_v, zbuf, acc):
    c = lax.axis_index("c")
    s = lax.axis_index("s")
    wid = c * NS + s

    def _fill(r, _):
        def _fj(j, _):
            ones_v[r, pl.ds(j * 16, 16)] = jnp.ones((16,), jnp.float32)
            zbuf[lax.rem(r, ZR), pl.ds(j * 16, 16)] = jnp.zeros(
                (16,), jnp.float32)
            return 0
        return lax.fori_loop(0, D // 16, _fj, 0)
    lax.fori_loop(0, K, _fill, 0)

    def _zc(k, _):
        pltpu.sync_copy(zbuf, acc.at[pl.ds(s * RPT + k * ZR, ZR)])
        return 0
    lax.fori_loop(0, RPT // ZR, _zc, 0)

    pltpu.sync_copy(colsr_hbm.at[wid], col_v)
    plsc.subcore_barrier()

    def _step(i, _):
        pltpu.sync_copy(ones_v, acc.at[col_v.at[i]], add=True)
        return 0
    lax.fori_loop(0, BINCH, _step, 0)

    plsc.subcore_barrier()
    pltpu.sync_copy(acc.at[pl.ds(s * RPT, RPT)],
                    out_hbm.at[c, pl.ds(s * RPT, RPT)])


def _scatter_body(y_hbm, rows_hbm, colsr_hbm, out_hbm,
                  row_v, col_v, gbuf, zbuf, acc, sem):
    c = lax.axis_index("c")
    s = lax.axis_index("s")
    wid = c * NS + s

    def _zr(r, _):
        def _zj(j, _):
            zbuf[r, pl.ds(j * 16, 16)] = jnp.zeros((16,), jnp.float32)
            return 0
        return lax.fori_loop(0, D // 16, _zj, 0)
    lax.fori_loop(0, ZR, _zr, 0)

    def _zc(k, _):
        pltpu.sync_copy(zbuf, acc.at[pl.ds(s * RPT + k * ZR, ZR)])
        return 0
    lax.fori_loop(0, RPT // ZR, _zc, 0)

    plsc.subcore_barrier()

    def _batch(t, _):
        pltpu.sync_copy(rows_hbm.at[wid, pl.ds(t * BCH, BCH)], row_v)
        pltpu.sync_copy(colsr_hbm.at[wid, pl.ds(t * BCH, BCH)], col_v)

        def _grp(g, _):
            def _fire(b, _):
                pltpu.async_copy(
                    y_hbm.at[row_v.at[g * NBUF + b]], gbuf.at[b], sem)
                return 0
            lax.fori_loop(0, NBUF, _fire, 0)

            def _drain(b, _):
                pltpu.make_async_copy(
                    y_hbm.at[row_v.at[g * NBUF + b]], gbuf.at[b], sem).wait()
                return 0
            lax.fori_loop(0, NBUF, _drain, 0)

            def _scat(b, _):
                pltpu.sync_copy(gbuf.at[b], acc.at[col_v.at[g * NBUF + b]],
                                add=True)
                return 0
            lax.fori_loop(0, NBUF, _scat, 0)
            return 0
        lax.fori_loop(0, BCH // NBUF, _grp, 0)
        return 0
    lax.fori_loop(0, BINCH // BCH, _batch, 0)

    plsc.subcore_barrier()
    pltpu.sync_copy(acc.at[pl.ds(s * RPT, RPT)],
                    out_hbm.at[c, pl.ds(s * RPT, RPT)])


@functools.lru_cache(maxsize=1)
def _sc_kernels():
    mesh = plsc.VectorSubcoreMesh(
        core_axis_name="c", subcore_axis_name="s",
        num_cores=NC, num_subcores=NS)
    bin_k = pl.kernel(
        _bin_body,
        out_type=[jax.ShapeDtypeStruct((NC, NS, BCAP), jnp.int32),
                  jax.ShapeDtypeStruct((NC, NS, BCAP), jnp.int32)],
        mesh=mesh,
        compiler_params=pltpu.CompilerParams(needs_layout_passes=False),
        scratch_types=[
            pltpu.VMEM((SCANB * K,), jnp.int32),
            pltpu.VMEM((SCANB * K,), jnp.int32),
            pltpu.VMEM((BCAP + 16,), jnp.int32),
            pltpu.VMEM((BCAP + 16,), jnp.int32),
        ],
    )
    deg_k = pl.kernel(
        _deg_body,
        out_type=jax.ShapeDtypeStruct((NC, HALF, D), jnp.float32),
        mesh=mesh,
        scratch_types=[
            pltpu.VMEM((BINCH, K), jnp.int32),
            pltpu.VMEM((K, D), jnp.float32),
            pltpu.VMEM((ZR, D), jnp.float32),
            pltpu.VMEM_SHARED((ACC_R, D), jnp.float32),
        ],
    )
    scat_k = pl.kernel(
        _scatter_body,
        out_type=jax.ShapeDtypeStruct((NC, HALF, D), jnp.float32),
        mesh=mesh,
        scratch_types=[
            pltpu.VMEM((BCH, K), jnp.int32),
            pltpu.VMEM((BCH, K), jnp.int32),
            pltpu.VMEM((NBUF, K, D), jnp.float32),
            pltpu.VMEM((ZR, D), jnp.float32),
            pltpu.VMEM_SHARED((ACC_R, D), jnp.float32),
            pltpu.SemaphoreType.DMA,
        ],
    )
    return bin_k, deg_k, scat_k


def _k1_body(deg_ref, x_ref, w1_ref, y1_ref, dinvb_ref):
    cnt = jnp.max(deg_ref[...], axis=1, keepdims=True)
    dinvb = jnp.broadcast_to(lax.rsqrt(cnt + 1.0), (D, D))
    xw = jnp.dot(x_ref[...], w1_ref[...], preferred_element_type=jnp.float32)
    y1_ref[...] = dinvb * xw
    dinvb_ref[...] = dinvb


def _k2_body(acc_ref, y1_ref, dinvb_ref, w2_ref, b1_ref, y2_ref):
    agg = acc_ref[...] + y1_ref[...]
    h1 = jnp.maximum(dinvb_ref[...] * agg + b1_ref[...], 0.0)
    y2_ref[...] = dinvb_ref[...] * jnp.dot(
        h1, w2_ref[...], preferred_element_type=jnp.float32)


def _k3_body(acc_ref, y2_ref, dinvb_ref, b2_ref, w3_ref, b3_ref, out_ref):
    agg = acc_ref[...] + y2_ref[...]
    h2 = jnp.maximum(dinvb_ref[...] * agg + b2_ref[...], 0.0)
    logits = jnp.dot(h2, w3_ref[...],
                     preferred_element_type=jnp.float32) + b3_ref[...]
    m = jnp.max(logits, axis=1, keepdims=True)
    e = jnp.exp(logits - m)
    out_ref[...] = e / jnp.sum(e, axis=1, keepdims=True)


_G = R_PAD // D

_blk_rows = pl.BlockSpec((D, D), lambda i: (i, 0))
_blk_full = pl.BlockSpec((D, D), lambda i: (0, 0))
_blk_bias = pl.BlockSpec((1, D), lambda i: (0, 0))

_k1 = pl.pallas_call(
    _k1_body,
    grid=(_G,),
    in_specs=[_blk_rows, _blk_rows, _blk_full],
    out_specs=[_blk_rows, _blk_rows],
    out_shape=[jax.ShapeDtypeStruct((R_PAD, D), jnp.float32),
               jax.ShapeDtypeStruct((R_PAD, D), jnp.float32)],
)

_k2 = pl.pallas_call(
    _k2_body,
    grid=(_G,),
    in_specs=[_blk_rows, _blk_rows, _blk_rows, _blk_full, _blk_bias],
    out_specs=_blk_rows,
    out_shape=jax.ShapeDtypeStruct((R_PAD, D), jnp.float32),
)

_k3 = pl.pallas_call(
    _k3_body,
    grid=(_G,),
    in_specs=[_blk_rows, _blk_rows, _blk_rows, _blk_bias, _blk_full,
              _blk_bias],
    out_specs=_blk_rows,
    out_shape=jax.ShapeDtypeStruct((R_PAD, D), jnp.float32),
)


def kernel(x, edge_index, W1, b1, W2, b2, W3, b3):
    row = edge_index[0].astype(jnp.int32)
    col = edge_index[1].astype(jnp.int32)
    pad_r = jnp.full((E_PAD - E,), N, jnp.int32)
    pad_c = jnp.full((E_PAD - E,), -1, jnp.int32)
    rows3 = jnp.concatenate([row, pad_r]).reshape(NS, CHT * K)
    cols3 = jnp.concatenate([col, pad_c]).reshape(NS, CHT * K)

    x_pad = jnp.concatenate(
        [x, jnp.zeros((R_PAD - N, D), jnp.float32)], axis=0)
    b1r = b1.reshape(1, D)
    b2r = b2.reshape(1, D)
    W3p = jnp.concatenate(
        [W3, jnp.zeros((D, D - NCLS), jnp.float32)], axis=1)
    b3p = jnp.concatenate(
        [b3, jnp.full((D - NCLS,), -1e30, jnp.float32)]).reshape(1, D)

    bin_kernel, deg_kernel, scatter_kernel = _sc_kernels()
    brows, bcols = bin_kernel(rows3, cols3)
    brows = brows.reshape(NC * NS, BINCH, K)
    bcols = bcols.reshape(NC * NS, BINCH, K)
    deg = deg_kernel(bcols).reshape(R_PAD, D)
    y1, dinvb = _k1(deg, x_pad, W1)
    acc1 = scatter_kernel(y1, brows, bcols).reshape(R_PAD, D)
    y2 = _k2(acc1, y1, dinvb, W2, b1r)
    acc2 = scatter_kernel(y2, brows, bcols).reshape(R_PAD, D)
    probs = _k3(acc2, y2, dinvb, b2r, W3p, b3p)
    return probs[:N, :NCLS]


# speedup vs baseline: 10.7393x; 1.1428x over previous
import functools

import jax
import jax.numpy as jnp
from jax import lax
from jax.experimental import pallas as pl
from jax.experimental.pallas import tpu as pltpu
from jax.experimental.pallas import tpu_sc as plsc

N = 10000
E = 320000
D = 128
NCLS = 40

NC = 2
NS = 16
K = 128
CHT = 160
E_PAD = NS * CHT * K
R_PAD = 10240
HALF = R_PAD // NC
ACC_R = HALF + K
ZR = 64
RPT = HALF // NS
GW = 2
BINCH = 96
BCAP = BINCH * K
BCH = 48
SCANB = 40


def _bin_body(rows_hbm, cols_hbm, brow_hbm, bcol_hbm,
              row_v, col_v, brow_b, bcol_b):
    c = lax.axis_index("c")
    s = lax.axis_index("s")
    lanes = lax.iota(jnp.int32, 16)

    def _pf(i, _):
        bcol_b[pl.ds(i * 16, 16)] = HALF + lax.rem(i, 8) * 16 + lanes
        brow_b[pl.ds(i * 16, 16)] = N + 16 + lax.rem(i, 14) * 16 + lanes
        return 0
    lax.fori_loop(0, BCAP // 16, _pf, 0)

    def _batch(t, off):
        pltpu.sync_copy(rows_hbm.at[s, pl.ds(t * SCANB * K, SCANB * K)],
                        row_v)
        pltpu.sync_copy(cols_hbm.at[s, pl.ds(t * SCANB * K, SCANB * K)],
                        col_v)

        def _vec(i, off):
            cv = col_v[pl.ds(i * 16, 16)]
            rv = row_v[pl.ds(i * 16, 16)]
            loc = cv - c * HALF
            msk = (loc >= 0) & (loc < HALF)
            key = jnp.where(msk, 0, 16) + lanes
            colsafe = jnp.where(msk, loc, HALF + lanes)
            rowsafe = jnp.where(msk, rv, N)
            bcol_b[pl.ds(off, 16)] = plsc.sort_key_val(key, colsafe)[1]
            brow_b[pl.ds(off, 16)] = plsc.sort_key_val(key, rowsafe)[1]
            n = plsc.all_reduce_population_count(msk)
            return lax.min(off + jnp.max(n), BCAP)
        return lax.fori_loop(0, SCANB * K // 16, _vec, off)
    lax.fori_loop(0, CHT // SCANB, _batch, 0)

    pltpu.sync_copy(brow_b.at[pl.ds(0, BCAP)], brow_hbm.at[c, s])
    pltpu.sync_copy(bcol_b.at[pl.ds(0, BCAP)], bcol_hbm.at[c, s])


def _deg_body(colsr_hbm, out_hbm, col_v, ones_v, zbuf, acc):
    c = lax.axis_index("c")
    s = lax.axis_index("s")
    wid = c * NS + s

    def _fill(r, _):
        def _fj(j, _):
            ones_v[r, pl.ds(j * 16, 16)] = jnp.ones((16,), jnp.float32)
            zbuf[lax.rem(r, ZR), pl.ds(j * 16, 16)] = jnp.zeros(
                (16,), jnp.float32)
            return 0
        return lax.fori_loop(0, D // 16, _fj, 0)
    lax.fori_loop(0, K, _fill, 0)

    def _zc(k, _):
        pltpu.sync_copy(zbuf, acc.at[pl.ds(s * RPT + k * ZR, ZR)])
        return 0
    lax.fori_loop(0, RPT // ZR, _zc, 0)

    pltpu.sync_copy(colsr_hbm.at[wid], col_v)
    plsc.subcore_barrier()

    def _step(i, _):
        pltpu.sync_copy(ones_v, acc.at[col_v.at[i]], add=True)
        return 0
    lax.fori_loop(0, BINCH, _step, 0)

    plsc.subcore_barrier()
    pltpu.sync_copy(acc.at[pl.ds(s * RPT, RPT)],
                    out_hbm.at[c, pl.ds(s * RPT, RPT)])


def _scatter_body(y_hbm, rows_hbm, colsr_hbm, out_hbm,
                  row_v, col_v, gbuf, zbuf, acc, sem):
    c = lax.axis_index("c")
    s = lax.axis_index("s")
    wid = c * NS + s

    def _zr(r, _):
        def _zj(j, _):
            zbuf[r, pl.ds(j * 16, 16)] = jnp.zeros((16,), jnp.float32)
            return 0
        return lax.fori_loop(0, D // 16, _zj, 0)
    lax.fori_loop(0, ZR, _zr, 0)

    def _zc(k, _):
        pltpu.sync_copy(zbuf, acc.at[pl.ds(s * RPT + k * ZR, ZR)])
        return 0
    lax.fori_loop(0, RPT // ZR, _zc, 0)

    plsc.subcore_barrier()

    def _fire(p, _):
        def _f(b, _):
            pltpu.async_copy(y_hbm.at[row_v.at[p * GW + b]],
                             gbuf.at[lax.rem(p * GW + b, 2 * GW)],
                             sem.at[lax.rem(p, 2)])
            return 0
        return lax.fori_loop(0, GW, _f, 0)

    def _drain_scat(p, _):
        def _d(b, _):
            pltpu.make_async_copy(
                y_hbm.at[row_v.at[p * GW + b]],
                gbuf.at[lax.rem(p * GW + b, 2 * GW)],
                sem.at[lax.rem(p, 2)]).wait()
            return 0
        lax.fori_loop(0, GW, _d, 0)

        def _s(b, _):
            pltpu.sync_copy(gbuf.at[lax.rem(p * GW + b, 2 * GW)],
                            acc.at[col_v.at[p * GW + b]], add=True)
            return 0
        return lax.fori_loop(0, GW, _s, 0)

    ngrp = BCH // GW

    def _batch(t, _):
        pltpu.sync_copy(rows_hbm.at[wid, pl.ds(t * BCH, BCH)], row_v)
        pltpu.sync_copy(colsr_hbm.at[wid, pl.ds(t * BCH, BCH)], col_v)
        _fire(0, 0)

        def _pipe(p, _):
            _fire(p + 1, 0)
            _drain_scat(p, 0)
            return 0
        lax.fori_loop(0, ngrp - 1, _pipe, 0)
        _drain_scat(ngrp - 1, 0)
        return 0
    lax.fori_loop(0, BINCH // BCH, _batch, 0)

    plsc.subcore_barrier()
    pltpu.sync_copy(acc.at[pl.ds(s * RPT, RPT)],
                    out_hbm.at[c, pl.ds(s * RPT, RPT)])


@functools.lru_cache(maxsize=1)
def _sc_kernels():
    mesh = plsc.VectorSubcoreMesh(
        core_axis_name="c", subcore_axis_name="s",
        num_cores=NC, num_subcores=NS)
    bin_k = pl.kernel(
        _bin_body,
        out_type=[jax.ShapeDtypeStruct((NC, NS, BCAP), jnp.int32),
                  jax.ShapeDtypeStruct((NC, NS, BCAP), jnp.int32)],
        mesh=mesh,
        compiler_params=pltpu.CompilerParams(needs_layout_passes=False),
        scratch_types=[
            pltpu.VMEM((SCANB * K,), jnp.int32),
            pltpu.VMEM((SCANB * K,), jnp.int32),
            pltpu.VMEM((BCAP + 16,), jnp.int32),
            pltpu.VMEM((BCAP + 16,), jnp.int32),
        ],
    )
    deg_k = pl.kernel(
        _deg_body,
        out_type=jax.ShapeDtypeStruct((NC, HALF, D), jnp.float32),
        mesh=mesh,
        scratch_types=[
            pltpu.VMEM((BINCH, K), jnp.int32),
            pltpu.VMEM((K, D), jnp.float32),
            pltpu.VMEM((ZR, D), jnp.float32),
            pltpu.VMEM_SHARED((ACC_R, D), jnp.float32),
        ],
    )
    scat_k = pl.kernel(
        _scatter_body,
        out_type=jax.ShapeDtypeStruct((NC, HALF, D), jnp.float32),
        mesh=mesh,
        scratch_types=[
            pltpu.VMEM((BCH, K), jnp.int32),
            pltpu.VMEM((BCH, K), jnp.int32),
            pltpu.VMEM((2 * GW, K, D), jnp.float32),
            pltpu.VMEM((ZR, D), jnp.float32),
            pltpu.VMEM_SHARED((ACC_R, D), jnp.float32),
            pltpu.SemaphoreType.DMA((2,)),
        ],
    )
    return bin_k, deg_k, scat_k


def _k1a_body(x_ref, w1_ref, xw_ref):
    xw_ref[...] = jnp.dot(x_ref[...], w1_ref[...],
                          preferred_element_type=jnp.float32)


def _k1_body(deg_ref, xw_ref, y1_ref, dinvb_ref):
    cnt = jnp.max(deg_ref[...], axis=1, keepdims=True)
    dinvb = jnp.broadcast_to(lax.rsqrt(cnt + 1.0), (D, D))
    y1_ref[...] = dinvb * xw_ref[...]
    dinvb_ref[...] = dinvb


def _k2_body(acc_ref, y1_ref, dinvb_ref, w2_ref, b1_ref, y2_ref):
    agg = acc_ref[...] + y1_ref[...]
    h1 = jnp.maximum(dinvb_ref[...] * agg + b1_ref[...], 0.0)
    y2_ref[...] = dinvb_ref[...] * jnp.dot(
        h1, w2_ref[...], preferred_element_type=jnp.float32)


def _k3_body(acc_ref, y2_ref, dinvb_ref, b2_ref, w3_ref, b3_ref, out_ref):
    agg = acc_ref[...] + y2_ref[...]
    h2 = jnp.maximum(dinvb_ref[...] * agg + b2_ref[...], 0.0)
    logits = jnp.dot(h2, w3_ref[...],
                     preferred_element_type=jnp.float32) + b3_ref[...]
    m = jnp.max(logits, axis=1, keepdims=True)
    e = jnp.exp(logits - m)
    out_ref[...] = e / jnp.sum(e, axis=1, keepdims=True)


_G = R_PAD // D

_blk_rows = pl.BlockSpec((D, D), lambda i: (i, 0))
_blk_full = pl.BlockSpec((D, D), lambda i: (0, 0))
_blk_bias = pl.BlockSpec((1, D), lambda i: (0, 0))

_k1a = pl.pallas_call(
    _k1a_body,
    grid=(_G,),
    in_specs=[_blk_rows, _blk_full],
    out_specs=_blk_rows,
    out_shape=jax.ShapeDtypeStruct((R_PAD, D), jnp.float32),
)

_k1 = pl.pallas_call(
    _k1_body,
    grid=(_G,),
    in_specs=[_blk_rows, _blk_rows],
    out_specs=[_blk_rows, _blk_rows],
    out_shape=[jax.ShapeDtypeStruct((R_PAD, D), jnp.float32),
               jax.ShapeDtypeStruct((R_PAD, D), jnp.float32)],
)

_k2 = pl.pallas_call(
    _k2_body,
    grid=(_G,),
    in_specs=[_blk_rows, _blk_rows, _blk_rows, _blk_full, _blk_bias],
    out_specs=_blk_rows,
    out_shape=jax.ShapeDtypeStruct((R_PAD, D), jnp.float32),
)

_k3 = pl.pallas_call(
    _k3_body,
    grid=(_G,),
    in_specs=[_blk_rows, _blk_rows, _blk_rows, _blk_bias, _blk_full,
              _blk_bias],
    out_specs=_blk_rows,
    out_shape=jax.ShapeDtypeStruct((R_PAD, D), jnp.float32),
)


def kernel(x, edge_index, W1, b1, W2, b2, W3, b3):
    row = edge_index[0].astype(jnp.int32)
    col = edge_index[1].astype(jnp.int32)
    pad_r = jnp.full((E_PAD - E,), N, jnp.int32)
    pad_c = jnp.full((E_PAD - E,), -1, jnp.int32)
    rows3 = jnp.concatenate([row, pad_r]).reshape(NS, CHT * K)
    cols3 = jnp.concatenate([col, pad_c]).reshape(NS, CHT * K)

    x_pad = jnp.concatenate(
        [x, jnp.zeros((R_PAD - N, D), jnp.float32)], axis=0)
    b1r = b1.reshape(1, D)
    b2r = b2.reshape(1, D)
    W3p = jnp.concatenate(
        [W3, jnp.zeros((D, D - NCLS), jnp.float32)], axis=1)
    b3p = jnp.concatenate(
        [b3, jnp.full((D - NCLS,), -1e30, jnp.float32)]).reshape(1, D)

    bin_kernel, deg_kernel, scatter_kernel = _sc_kernels()
    brows, bcols = bin_kernel(rows3, cols3)
    brows = brows.reshape(NC * NS, BINCH, K)
    bcols = bcols.reshape(NC * NS, BINCH, K)
    deg = deg_kernel(bcols).reshape(R_PAD, D)
    xw1 = _k1a(x_pad, W1)
    y1, dinvb = _k1(deg, xw1)
    acc1 = scatter_kernel(y1, brows, bcols).reshape(R_PAD, D)
    y2 = _k2(acc1, y1, dinvb, W2, b1r)
    acc2 = scatter_kernel(y2, brows, bcols).reshape(R_PAD, D)
    probs = _k3(acc2, y2, dinvb, b2r, W3p, b3p)
    return probs[:N, :NCLS]


# speedup vs baseline: 12.1799x; 1.1341x over previous
import functools

import jax
import jax.numpy as jnp
from jax import lax
from jax.experimental import pallas as pl
from jax.experimental.pallas import tpu as pltpu
from jax.experimental.pallas import tpu_sc as plsc

N = 10000
E = 320000
D = 128
NCLS = 40

NC = 2
NS = 16
K = 128
CHT = 160
E_PAD = NS * CHT * K
R_PAD = 10240
HALF = R_PAD // NC
ACC_R = HALF + K
ZR = 64
RPT = HALF // NS
GW = 2
BINCH = 96
BCAP = BINCH * K
BCH = 48
SCANB = 40


def _bin_body(rows_hbm, cols_hbm, brow_hbm, bcol_hbm, bcnt_hbm,
              row_v, col_v, brow_b, bcol_b, cnt_v):
    c = lax.axis_index("c")
    s = lax.axis_index("s")
    lanes = lax.iota(jnp.int32, 16)

    def _pf(i, _):
        bcol_b[pl.ds(i * 16, 16)] = HALF + lax.rem(i, 8) * 16 + lanes
        brow_b[pl.ds(i * 16, 16)] = N + 16 + lax.rem(i, 14) * 16 + lanes
        return 0
    lax.fori_loop(0, BCAP // 16, _pf, 0)

    def _batch(t, off):
        pltpu.sync_copy(rows_hbm.at[s, pl.ds(t * SCANB * K, SCANB * K)],
                        row_v)
        pltpu.sync_copy(cols_hbm.at[s, pl.ds(t * SCANB * K, SCANB * K)],
                        col_v)

        def _vec(i, off):
            cv = col_v[pl.ds(i * 16, 16)]
            rv = row_v[pl.ds(i * 16, 16)]
            loc = cv - c * HALF
            msk = (loc >= 0) & (loc < HALF)
            key = jnp.where(msk, 0, 16) + lanes
            colsafe = jnp.where(msk, loc, HALF + lanes)
            rowsafe = jnp.where(msk, rv, N)
            bcol_b[pl.ds(off, 16)] = plsc.sort_key_val(key, colsafe)[1]
            brow_b[pl.ds(off, 16)] = plsc.sort_key_val(key, rowsafe)[1]
            n = plsc.all_reduce_population_count(msk)
            return lax.min(off + jnp.max(n), BCAP)
        return lax.fori_loop(0, SCANB * K // 16, _vec, off)
    off = lax.fori_loop(0, CHT // SCANB, _batch, 0)

    cnt_v[pl.ds(0, 16)] = jnp.full((16,), 0, jnp.int32) + off
    pltpu.sync_copy(brow_b.at[pl.ds(0, BCAP)], brow_hbm.at[c, s])
    pltpu.sync_copy(bcol_b.at[pl.ds(0, BCAP)], bcol_hbm.at[c, s])
    pltpu.sync_copy(cnt_v, bcnt_hbm.at[c, s])


def _deg_body(colsr_hbm, bcnt_hbm, out_hbm, col_v, ones_v, zbuf, cnt_v, acc):
    c = lax.axis_index("c")
    s = lax.axis_index("s")
    wid = c * NS + s
    pltpu.sync_copy(bcnt_hbm.at[c, s], cnt_v)
    nch = (cnt_v[pl.ds(0, 16)][0] + K - 1) // K

    def _fill(r, _):
        def _fj(j, _):
            ones_v[r, pl.ds(j * 16, 16)] = jnp.ones((16,), jnp.float32)
            zbuf[lax.rem(r, ZR), pl.ds(j * 16, 16)] = jnp.zeros(
                (16,), jnp.float32)
            return 0
        return lax.fori_loop(0, D // 16, _fj, 0)
    lax.fori_loop(0, K, _fill, 0)

    def _zc(k, _):
        pltpu.sync_copy(zbuf, acc.at[pl.ds(s * RPT + k * ZR, ZR)])
        return 0
    lax.fori_loop(0, RPT // ZR, _zc, 0)

    pltpu.sync_copy(colsr_hbm.at[wid], col_v)
    plsc.subcore_barrier()

    def _step(i, _):
        @pl.when(i < nch)
        def _():
            pltpu.sync_copy(ones_v, acc.at[col_v.at[i]], add=True)
        return 0
    lax.fori_loop(0, BINCH, _step, 0)

    plsc.subcore_barrier()
    pltpu.sync_copy(acc.at[pl.ds(s * RPT, RPT)],
                    out_hbm.at[c, pl.ds(s * RPT, RPT)])


def _scatter_body(y_hbm, rows_hbm, colsr_hbm, bcnt_hbm, out_hbm,
                  row_v, col_v, gbuf, zbuf, cnt_v, acc, sem):
    c = lax.axis_index("c")
    s = lax.axis_index("s")
    wid = c * NS + s
    pltpu.sync_copy(bcnt_hbm.at[c, s], cnt_v)
    nch = (cnt_v[pl.ds(0, 16)][0] + K - 1) // K

    def _zr(r, _):
        def _zj(j, _):
            zbuf[r, pl.ds(j * 16, 16)] = jnp.zeros((16,), jnp.float32)
            return 0
        return lax.fori_loop(0, D // 16, _zj, 0)
    lax.fori_loop(0, ZR, _zr, 0)

    def _zc(k, _):
        pltpu.sync_copy(zbuf, acc.at[pl.ds(s * RPT + k * ZR, ZR)])
        return 0
    lax.fori_loop(0, RPT // ZR, _zc, 0)

    plsc.subcore_barrier()

    def _fire(t, p, _):
        def _f(b, _):
            @pl.when(t * BCH + p * GW + b < nch)
            def _():
                pltpu.async_copy(y_hbm.at[row_v.at[p * GW + b]],
                                 gbuf.at[lax.rem(p * GW + b, 2 * GW)],
                                 sem.at[lax.rem(p, 2)])
            return 0
        return lax.fori_loop(0, GW, _f, 0)

    def _drain_scat(t, p, _):
        def _d(b, _):
            @pl.when(t * BCH + p * GW + b < nch)
            def _():
                pltpu.make_async_copy(
                    y_hbm.at[row_v.at[p * GW + b]],
                    gbuf.at[lax.rem(p * GW + b, 2 * GW)],
                    sem.at[lax.rem(p, 2)]).wait()
            return 0
        lax.fori_loop(0, GW, _d, 0)

        def _s(b, _):
            @pl.when(t * BCH + p * GW + b < nch)
            def _():
                pltpu.sync_copy(gbuf.at[lax.rem(p * GW + b, 2 * GW)],
                                acc.at[col_v.at[p * GW + b]], add=True)
            return 0
        return lax.fori_loop(0, GW, _s, 0)

    ngrp = BCH // GW

    def _batch(t, _):
        pltpu.sync_copy(rows_hbm.at[wid, pl.ds(t * BCH, BCH)], row_v)
        pltpu.sync_copy(colsr_hbm.at[wid, pl.ds(t * BCH, BCH)], col_v)
        _fire(t, 0, 0)

        def _pipe(p, _):
            _fire(t, p + 1, 0)
            _drain_scat(t, p, 0)
            return 0
        lax.fori_loop(0, ngrp - 1, _pipe, 0)
        _drain_scat(t, ngrp - 1, 0)
        return 0
    lax.fori_loop(0, BINCH // BCH, _batch, 0)

    plsc.subcore_barrier()
    pltpu.sync_copy(acc.at[pl.ds(s * RPT, RPT)],
                    out_hbm.at[c, pl.ds(s * RPT, RPT)])


@functools.lru_cache(maxsize=1)
def _sc_kernels():
    mesh = plsc.VectorSubcoreMesh(
        core_axis_name="c", subcore_axis_name="s",
        num_cores=NC, num_subcores=NS)
    bin_k = pl.kernel(
        _bin_body,
        out_type=[jax.ShapeDtypeStruct((NC, NS, BCAP), jnp.int32),
                  jax.ShapeDtypeStruct((NC, NS, BCAP), jnp.int32),
                  jax.ShapeDtypeStruct((NC, NS, 16), jnp.int32)],
        mesh=mesh,
        compiler_params=pltpu.CompilerParams(needs_layout_passes=False),
        scratch_types=[
            pltpu.VMEM((SCANB * K,), jnp.int32),
            pltpu.VMEM((SCANB * K,), jnp.int32),
            pltpu.VMEM((BCAP + 16,), jnp.int32),
            pltpu.VMEM((BCAP + 16,), jnp.int32),
            pltpu.VMEM((16,), jnp.int32),
        ],
    )
    deg_k = pl.kernel(
        _deg_body,
        out_type=jax.ShapeDtypeStruct((NC, HALF, D), jnp.float32),
        mesh=mesh,
        scratch_types=[
            pltpu.VMEM((BINCH, K), jnp.int32),
            pltpu.VMEM((K, D), jnp.float32),
            pltpu.VMEM((ZR, D), jnp.float32),
            pltpu.VMEM((16,), jnp.int32),
            pltpu.VMEM_SHARED((ACC_R, D), jnp.float32),
        ],
    )
    scat_k = pl.kernel(
        _scatter_body,
        out_type=jax.ShapeDtypeStruct((NC, HALF, D), jnp.float32),
        mesh=mesh,
        scratch_types=[
            pltpu.VMEM((BCH, K), jnp.int32),
            pltpu.VMEM((BCH, K), jnp.int32),
            pltpu.VMEM((2 * GW, K, D), jnp.float32),
            pltpu.VMEM((ZR, D), jnp.float32),
            pltpu.VMEM((16,), jnp.int32),
            pltpu.VMEM_SHARED((ACC_R, D), jnp.float32),
            pltpu.SemaphoreType.DMA((2,)),
        ],
    )
    return bin_k, deg_k, scat_k


def _k1a_body(x_ref, w1_ref, xw_ref):
    xw_ref[...] = jnp.dot(x_ref[...], w1_ref[...],
                          preferred_element_type=jnp.float32)


def _k1_body(deg_ref, xw_ref, y1_ref, dinvb_ref):
    cnt = jnp.max(deg_ref[...], axis=1, keepdims=True)
    dinvb = jnp.broadcast_to(lax.rsqrt(cnt + 1.0), (D, D))
    y1_ref[...] = dinvb * xw_ref[...]
    dinvb_ref[...] = dinvb


def _k2_body(acc_ref, y1_ref, dinvb_ref, w2_ref, b1_ref, y2_ref):
    agg = acc_ref[...] + y1_ref[...]
    h1 = jnp.maximum(dinvb_ref[...] * agg + b1_ref[...], 0.0)
    y2_ref[...] = dinvb_ref[...] * jnp.dot(
        h1, w2_ref[...], preferred_element_type=jnp.float32)


def _k3_body(acc_ref, y2_ref, dinvb_ref, b2_ref, w3_ref, b3_ref, out_ref):
    agg = acc_ref[...] + y2_ref[...]
    h2 = jnp.maximum(dinvb_ref[...] * agg + b2_ref[...], 0.0)
    logits = jnp.dot(h2, w3_ref[...],
                     preferred_element_type=jnp.float32) + b3_ref[...]
    m = jnp.max(logits, axis=1, keepdims=True)
    e = jnp.exp(logits - m)
    out_ref[...] = e / jnp.sum(e, axis=1, keepdims=True)


_G = R_PAD // D

_blk_rows = pl.BlockSpec((D, D), lambda i: (i, 0))
_blk_full = pl.BlockSpec((D, D), lambda i: (0, 0))
_blk_bias = pl.BlockSpec((1, D), lambda i: (0, 0))

_k1a = pl.pallas_call(
    _k1a_body,
    grid=(_G,),
    in_specs=[_blk_rows, _blk_full],
    out_specs=_blk_rows,
    out_shape=jax.ShapeDtypeStruct((R_PAD, D), jnp.float32),
)

_k1 = pl.pallas_call(
    _k1_body,
    grid=(_G,),
    in_specs=[_blk_rows, _blk_rows],
    out_specs=[_blk_rows, _blk_rows],
    out_shape=[jax.ShapeDtypeStruct((R_PAD, D), jnp.float32),
               jax.ShapeDtypeStruct((R_PAD, D), jnp.float32)],
)

_k2 = pl.pallas_call(
    _k2_body,
    grid=(_G,),
    in_specs=[_blk_rows, _blk_rows, _blk_rows, _blk_full, _blk_bias],
    out_specs=_blk_rows,
    out_shape=jax.ShapeDtypeStruct((R_PAD, D), jnp.float32),
)

_k3 = pl.pallas_call(
    _k3_body,
    grid=(_G,),
    in_specs=[_blk_rows, _blk_rows, _blk_rows, _blk_bias, _blk_full,
              _blk_bias],
    out_specs=_blk_rows,
    out_shape=jax.ShapeDtypeStruct((R_PAD, D), jnp.float32),
)


def kernel(x, edge_index, W1, b1, W2, b2, W3, b3):
    row = edge_index[0].astype(jnp.int32)
    col = edge_index[1].astype(jnp.int32)
    pad_r = jnp.full((E_PAD - E,), N, jnp.int32)
    pad_c = jnp.full((E_PAD - E,), -1, jnp.int32)
    rows3 = jnp.concatenate([row, pad_r]).reshape(NS, CHT * K)
    cols3 = jnp.concatenate([col, pad_c]).reshape(NS, CHT * K)

    x_pad = jnp.concatenate(
        [x, jnp.zeros((R_PAD - N, D), jnp.float32)], axis=0)
    b1r = b1.reshape(1, D)
    b2r = b2.reshape(1, D)
    W3p = jnp.concatenate(
        [W3, jnp.zeros((D, D - NCLS), jnp.float32)], axis=1)
    b3p = jnp.concatenate(
        [b3, jnp.full((D - NCLS,), -1e30, jnp.float32)]).reshape(1, D)

    bin_kernel, deg_kernel, scatter_kernel = _sc_kernels()
    brows, bcols, bcnt = bin_kernel(rows3, cols3)
    brows = brows.reshape(NC * NS, BINCH, K)
    bcols = bcols.reshape(NC * NS, BINCH, K)
    deg = deg_kernel(bcols, bcnt).reshape(R_PAD, D)
    xw1 = _k1a(x_pad, W1)
    y1, dinvb = _k1(deg, xw1)
    acc1 = scatter_kernel(y1, brows, bcols, bcnt).reshape(R_PAD, D)
    y2 = _k2(acc1, y1, dinvb, W2, b1r)
    acc2 = scatter_kernel(y2, brows, bcols, bcnt).reshape(R_PAD, D)
    probs = _k3(acc2, y2, dinvb, b2r, W3p, b3p)
    return probs[:N, :NCLS]
